# baseline (speedup 1.0000x reference)
import functools

import jax
import jax.numpy as jnp
from jax import lax
from jax.experimental import pallas as pl
from jax.experimental.pallas import tpu as pltpu
from jax.experimental.pallas import tpu_sc as plsc

L = 16
RSHIFT = 17
NB = 1 << (31 - RSHIFT)
NCHUNK = NB // L
TOPK_FRACTION = 0.1


def _bcast(x, dtype=jnp.int32):
    return lax.broadcast_in_dim(lax.convert_element_type(x, dtype), (L,), ())


@functools.cache
def _build(B, N):
    info = plsc.get_sparse_core_info()
    NC, NS = info.num_cores, info.num_subcores
    NW = NC * NS
    assert B % NW == 0, (B, NW)
    rows_per_w = B // NW
    k_top = int(TOPK_FRACTION * N)
    n_chunks = N // L

    mesh = plsc.VectorSubcoreMesh(core_axis_name="c", subcore_axis_name="s")

    @functools.partial(
        pl.kernel,
        out_type=jax.ShapeDtypeStruct((B, N), jnp.float32),
        mesh=mesh,
        compiler_params=pltpu.CompilerParams(needs_layout_passes=False),
        scratch_types=[
            pltpu.VMEM((N,), jnp.float32),
            pltpu.VMEM((N,), jnp.float32),
            pltpu.VMEM((N,), jnp.float32),
            pltpu.VMEM((NB,), jnp.int32),
            pltpu.VMEM((NCHUNK,), jnp.int32),
            pltpu.VMEM((NCHUNK // L,), jnp.int32),
            pltpu.VMEM((L,), jnp.int32),
            pltpu.SemaphoreType.DMA,
            pltpu.SemaphoreType.DMA,
            pltpu.SemaphoreType.DMA,
            pltpu.SemaphoreType.DMA,
            pltpu.SemaphoreType.DMA,
        ],
    )
    def sc_kernel(x_hbm, thr_hbm, out_hbm, row_a, row_b, thr_v, hist_v,
                  lvl1_v, lvl2_v, lvl3_v, sem_a, sem_b, sem_t, sem_oa,
                  sem_ob):
        wid = lax.axis_index("s") * NC + lax.axis_index("c")
        ones = jnp.ones((L,), jnp.int32)
        zeros = jnp.zeros((L,), jnp.int32)
        lane = lax.iota(jnp.int32, L)
        lane0 = lane == 0
        rows = [wid * rows_per_w + r for r in range(rows_per_w)]
        bufs = [row_a, row_b]
        in_sems = [sem_a, sem_b]
        out_sems = [sem_oa, sem_ob]

        in_copies = [
            pltpu.async_copy(x_hbm.at[rows[r]], bufs[r], in_sems[r])
            for r in range(rows_per_w)
        ]
        thr_copy = pltpu.async_copy(thr_hbm, thr_v, sem_t)
        out_copies = []

        lvl3_v[pl.ds(0, L)] = zeros

        @plsc.parallel_loop(0, NCHUNK, unroll=8)
        def _(i):
            hist_v[pl.ds(i * L, L)] = zeros

        def drill(vec, base_above, k_need):
            rh = jnp.flip(vec)
            suffix = plsc.cumsum(rh) + base_above
            m = suffix >= k_need
            first = m & (plsc.cumsum(m.astype(jnp.int32)) == 1)
            j = jnp.max(jnp.where(first, (L - 1) - lane, jnp.int32(-1)))
            above = jnp.max(jnp.where(first, suffix - rh, jnp.int32(-1)))
            return j, above

        def hist_scan(k_need, re_zero):
            @plsc.parallel_loop(0, NCHUNK, unroll=4)
            def _(c):
                t = jnp.sum(hist_v[pl.ds(c * L, L)])
                plsc.store_scatter(lvl1_v, [_bcast(c)], _bcast(t), mask=lane0)

            @plsc.parallel_loop(0, NCHUNK // L, unroll=4)
            def _(s):
                t = jnp.sum(lvl1_v[pl.ds(s * L, L)])
                plsc.store_scatter(lvl2_v, [_bcast(s)], _bcast(t), mask=lane0)

            @plsc.parallel_loop(0, NCHUNK // L // L)
            def _(u):
                t = jnp.sum(lvl2_v[pl.ds(u * L, L)])
                plsc.store_scatter(lvl3_v, [_bcast(u)], _bcast(t), mask=lane0)

            s3, above3 = drill(lvl3_v[pl.ds(0, L)], jnp.int32(0), k_need)
            s2, above2 = drill(lvl2_v[pl.ds(s3 * L, L)], above3, k_need)
            c2 = s3 * L + s2
            s1, above1 = drill(lvl1_v[pl.ds(c2 * L, L)], above2, k_need)
            c1 = c2 * L + s1
            b0, _ = drill(hist_v[pl.ds(c1 * L, L)], above1, k_need)
            bstar = c1 * L + b0

            if re_zero:
                @plsc.parallel_loop(0, NCHUNK, unroll=8)
                def _(i):
                    hist_v[pl.ds(i * L, L)] = zeros

            return bstar

        for r in range(rows_per_w):
            row_v = bufs[r]
            in_copies[r].wait()

            @plsc.parallel_loop(0, n_chunks, unroll=8)
            def _(i):
                v = row_v[pl.ds(i * L, L)]
                bits = lax.bitcast_convert_type(jnp.abs(v), jnp.int32)
                bkt = lax.shift_right_logical(bits, RSHIFT)
                cnt, lastm = plsc.scan_count(bkt)
                plsc.addupdate_scatter(hist_v, [bkt], cnt, mask=lastm)

            bstar = hist_scan(jnp.int32(k_top), re_zero=r < rows_per_w - 1)
            cutoff = lax.shift_left(bstar, RSHIFT)

            if r == 0:
                thr_copy.wait()

            @plsc.parallel_loop(0, n_chunks, unroll=8)
            def _(i):
                v = row_v[pl.ds(i * L, L)]
                vb = lax.bitcast_convert_type(v, jnp.int32)
                ab = vb & jnp.int32(0x7FFFFFFF)
                a = lax.bitcast_convert_type(ab, jnp.float32)
                m = jnp.maximum(a - thr_v[pl.ds(i * L, L)], jnp.float32(0.0))
                soft_b = (vb & jnp.int32(-0x80000000)) | \
                    lax.bitcast_convert_type(m, jnp.int32)
                out_b = jnp.where(ab >= cutoff, vb, soft_b)
                row_v[pl.ds(i * L, L)] = lax.bitcast_convert_type(
                    out_b, jnp.float32)

            out_copies.append(
                pltpu.async_copy(row_v, out_hbm.at[rows[r]], out_sems[r]))

        for c in out_copies:
            c.wait()

    return sc_kernel


def kernel(x, threshold):
    B, N = x.shape
    return _build(B, N)(x, threshold)


# speedup vs baseline: 1.0506x; 1.0506x over previous
import functools

import jax
import jax.numpy as jnp
from jax import lax
from jax.experimental import pallas as pl
from jax.experimental.pallas import tpu as pltpu
from jax.experimental.pallas import tpu_sc as plsc

L = 16
RSHIFT = 17
NB = 1 << (31 - RSHIFT)
NCHUNK = NB // L
TOPK_FRACTION = 0.1


def _bcast(x, dtype=jnp.int32):
    return lax.broadcast_in_dim(lax.convert_element_type(x, dtype), (L,), ())


@functools.cache
def _build(B, N):
    info = plsc.get_sparse_core_info()
    NC, NS = info.num_cores, info.num_subcores
    NW = NC * NS
    assert B % NW == 0, (B, NW)
    rows_per_w = B // NW
    k_top = int(TOPK_FRACTION * N)
    n_chunks = N // L

    mesh = plsc.VectorSubcoreMesh(core_axis_name="c", subcore_axis_name="s")

    @functools.partial(
        pl.kernel,
        out_type=jax.ShapeDtypeStruct((B, N), jnp.float32),
        mesh=mesh,
        compiler_params=pltpu.CompilerParams(needs_layout_passes=False),
        scratch_types=[
            pltpu.VMEM((N,), jnp.float32),
            pltpu.VMEM((N,), jnp.float32),
            pltpu.VMEM((N,), jnp.float32),
            pltpu.VMEM((NB,), jnp.int32),
            pltpu.VMEM((NCHUNK,), jnp.int32),
            pltpu.VMEM((NCHUNK // L,), jnp.int32),
            pltpu.VMEM((L,), jnp.int32),
        ] + [pltpu.SemaphoreType.DMA] * 9,
    )
    def sc_kernel(x_hbm, thr_hbm, out_hbm, row_a, row_b, thr_v, hist_v,
                  lvl1_v, lvl2_v, lvl3_v, *sems):
        wid = lax.axis_index("s") * NC + lax.axis_index("c")
        ones = jnp.ones((L,), jnp.int32)
        zeros = jnp.zeros((L,), jnp.int32)
        lane = lax.iota(jnp.int32, L)
        lane0 = lane == 0
        rows = [wid * rows_per_w + r for r in range(rows_per_w)]
        bufs = [row_a, row_b]
        NH = N // 2
        in_sems = [[sems[2 * r + h] for h in range(2)]
                   for r in range(rows_per_w)]
        out_sems = [[sems[4 + 2 * r + h] for h in range(2)]
                    for r in range(rows_per_w)]
        sem_t = sems[8]

        in_copies = [
            [pltpu.async_copy(x_hbm.at[rows[r], pl.ds(h * NH, NH)],
                              bufs[r].at[pl.ds(h * NH, NH)], in_sems[r][h])
             for h in range(2)]
            for r in range(rows_per_w)
        ]
        thr_copy = pltpu.async_copy(thr_hbm, thr_v, sem_t)
        out_copies = []

        lvl3_v[pl.ds(0, L)] = zeros

        @plsc.parallel_loop(0, NCHUNK, unroll=8)
        def _(i):
            hist_v[pl.ds(i * L, L)] = zeros

        def drill(vec, base_above, k_need):
            rh = jnp.flip(vec)
            suffix = plsc.cumsum(rh) + base_above
            m = suffix >= k_need
            first = m & (plsc.cumsum(m.astype(jnp.int32)) == 1)
            j = jnp.max(jnp.where(first, (L - 1) - lane, jnp.int32(-1)))
            above = jnp.max(jnp.where(first, suffix - rh, jnp.int32(-1)))
            return j, above

        def hist_scan(k_need, re_zero):
            @plsc.parallel_loop(0, NCHUNK, unroll=4)
            def _(c):
                t = jnp.sum(hist_v[pl.ds(c * L, L)])
                plsc.store_scatter(lvl1_v, [_bcast(c)], _bcast(t), mask=lane0)

            @plsc.parallel_loop(0, NCHUNK // L, unroll=4)
            def _(s):
                t = jnp.sum(lvl1_v[pl.ds(s * L, L)])
                plsc.store_scatter(lvl2_v, [_bcast(s)], _bcast(t), mask=lane0)

            @plsc.parallel_loop(0, NCHUNK // L // L)
            def _(u):
                t = jnp.sum(lvl2_v[pl.ds(u * L, L)])
                plsc.store_scatter(lvl3_v, [_bcast(u)], _bcast(t), mask=lane0)

            s3, above3 = drill(lvl3_v[pl.ds(0, L)], jnp.int32(0), k_need)
            s2, above2 = drill(lvl2_v[pl.ds(s3 * L, L)], above3, k_need)
            c2 = s3 * L + s2
            s1, above1 = drill(lvl1_v[pl.ds(c2 * L, L)], above2, k_need)
            c1 = c2 * L + s1
            b0, _ = drill(hist_v[pl.ds(c1 * L, L)], above1, k_need)
            bstar = c1 * L + b0

            if re_zero:
                @plsc.parallel_loop(0, NCHUNK, unroll=8)
                def _(i):
                    hist_v[pl.ds(i * L, L)] = zeros

            return bstar

        for r in range(rows_per_w):
            row_v = bufs[r]

            for h in range(2):
                in_copies[r][h].wait()

                @plsc.parallel_loop(h * (n_chunks // 2),
                                    (h + 1) * (n_chunks // 2), unroll=8)
                def _(i):
                    v = row_v[pl.ds(i * L, L)]
                    bits = lax.bitcast_convert_type(jnp.abs(v), jnp.int32)
                    plsc.addupdate_scatter(
                        hist_v, [lax.shift_right_logical(bits, RSHIFT)], ones)

            bstar = hist_scan(jnp.int32(k_top), re_zero=r < rows_per_w - 1)
            cutoff = lax.shift_left(bstar, RSHIFT)

            if r == 0:
                thr_copy.wait()

            for h in range(2):
                @plsc.parallel_loop(h * (n_chunks // 2),
                                    (h + 1) * (n_chunks // 2), unroll=8)
                def _(i):
                    v = row_v[pl.ds(i * L, L)]
                    vb = lax.bitcast_convert_type(v, jnp.int32)
                    ab = vb & jnp.int32(0x7FFFFFFF)
                    a = lax.bitcast_convert_type(ab, jnp.float32)
                    m = jnp.maximum(a - thr_v[pl.ds(i * L, L)],
                                    jnp.float32(0.0))
                    soft_b = (vb & jnp.int32(-0x80000000)) | \
                        lax.bitcast_convert_type(m, jnp.int32)
                    out_b = jnp.where(ab >= cutoff, vb, soft_b)
                    row_v[pl.ds(i * L, L)] = lax.bitcast_convert_type(
                        out_b, jnp.float32)

                out_copies.append(
                    pltpu.async_copy(bufs[r].at[pl.ds(h * NH, NH)],
                                     out_hbm.at[rows[r], pl.ds(h * NH, NH)],
                                     out_sems[r][h]))

        for c in out_copies:
            c.wait()

    return sc_kernel


def kernel(x, threshold):
    B, N = x.shape
    return _build(B, N)(x, threshold)


# speedup vs baseline: 1.0590x; 1.0080x over previous
import functools

import jax
import jax.numpy as jnp
from jax import lax
from jax.experimental import pallas as pl
from jax.experimental.pallas import tpu as pltpu
from jax.experimental.pallas import tpu_sc as plsc

L = 16
RSHIFT = 17
NB = 1 << (31 - RSHIFT)
NCHUNK = NB // L
TOPK_FRACTION = 0.1


def _bcast(x, dtype=jnp.int32):
    return lax.broadcast_in_dim(lax.convert_element_type(x, dtype), (L,), ())


@functools.cache
def _build(B, N):
    info = plsc.get_sparse_core_info()
    NC, NS = info.num_cores, info.num_subcores
    NW = NC * NS
    assert B % NW == 0, (B, NW)
    rows_per_w = B // NW
    assert rows_per_w == 2, rows_per_w
    k_top = int(TOPK_FRACTION * N)
    n_chunks = N // L

    mesh = plsc.VectorSubcoreMesh(core_axis_name="c", subcore_axis_name="s")

    @functools.partial(
        pl.kernel,
        out_type=jax.ShapeDtypeStruct((B, N), jnp.float32),
        mesh=mesh,
        compiler_params=pltpu.CompilerParams(needs_layout_passes=False),
        scratch_types=[
            pltpu.VMEM((N,), jnp.float32),
            pltpu.VMEM((N,), jnp.float32),
            pltpu.VMEM((N,), jnp.float32),
            pltpu.VMEM((NB,), jnp.int32),
            pltpu.VMEM((NCHUNK,), jnp.int32),
            pltpu.VMEM((NCHUNK // L,), jnp.int32),
            pltpu.VMEM((L,), jnp.int32),
        ] + [pltpu.SemaphoreType.DMA] * 9,
    )
    def sc_kernel(x_hbm, thr_hbm, out_hbm, row_a, row_b, thr_v, hist_v,
                  lvl1_v, lvl2_v, lvl3_v, *sems):
        wid = lax.axis_index("s") * NC + lax.axis_index("c")
        ones = jnp.ones((L,), jnp.int32)
        zeros = jnp.zeros((L,), jnp.int32)
        lane = lax.iota(jnp.int32, L)
        lane0 = lane == 0
        rows = [wid * rows_per_w + r for r in range(rows_per_w)]
        bufs = [row_a, row_b]
        NH = N // 2
        in_sems = [[sems[2 * r + h] for h in range(2)]
                   for r in range(rows_per_w)]
        out_sems = [[sems[4 + 2 * r + h] for h in range(2)]
                    for r in range(rows_per_w)]
        sem_t = sems[8]

        in_copies = [
            [pltpu.async_copy(x_hbm.at[rows[r], pl.ds(h * NH, NH)],
                              bufs[r].at[pl.ds(h * NH, NH)], in_sems[r][h])
             for h in range(2)]
            for r in range(rows_per_w)
        ]
        thr_copy = pltpu.async_copy(thr_hbm, thr_v, sem_t)
        out_copies = []

        lvl3_v[pl.ds(0, L)] = zeros

        @plsc.parallel_loop(0, NCHUNK, unroll=8)
        def _(i):
            hist_v[pl.ds(i * L, L)] = zeros

        def drill(vec, base_above, k_need):
            rh = jnp.flip(vec)
            suffix = plsc.cumsum(rh) + base_above
            m = suffix >= k_need
            first = m & (plsc.cumsum(m.astype(jnp.int32)) == 1)
            j = jnp.max(jnp.where(first, (L - 1) - lane, jnp.int32(-1)))
            above = jnp.max(jnp.where(first, suffix - rh, jnp.int32(-1)))
            return j, above

        def hist_scan(k_need, re_zero):
            @plsc.parallel_loop(0, NCHUNK, unroll=4)
            def _(c):
                t = jnp.sum(hist_v[pl.ds(c * L, L)])
                plsc.store_scatter(lvl1_v, [_bcast(c)], _bcast(t), mask=lane0)

            @plsc.parallel_loop(0, NCHUNK // L, unroll=4)
            def _(s):
                t = jnp.sum(lvl1_v[pl.ds(s * L, L)])
                plsc.store_scatter(lvl2_v, [_bcast(s)], _bcast(t), mask=lane0)

            @plsc.parallel_loop(0, NCHUNK // L // L)
            def _(u):
                t = jnp.sum(lvl2_v[pl.ds(u * L, L)])
                plsc.store_scatter(lvl3_v, [_bcast(u)], _bcast(t), mask=lane0)

            s3, above3 = drill(lvl3_v[pl.ds(0, L)], jnp.int32(0), k_need)
            s2, above2 = drill(lvl2_v[pl.ds(s3 * L, L)], above3, k_need)
            c2 = s3 * L + s2
            s1, above1 = drill(lvl1_v[pl.ds(c2 * L, L)], above2, k_need)
            c1 = c2 * L + s1
            b0, _ = drill(hist_v[pl.ds(c1 * L, L)], above1, k_need)
            bstar = c1 * L + b0

            if re_zero:
                @plsc.parallel_loop(0, NCHUNK, unroll=8)
                def _(i):
                    hist_v[pl.ds(i * L, L)] = zeros

            return bstar

        def hist_half(buf, h):
            @plsc.parallel_loop(h * (n_chunks // 2),
                                (h + 1) * (n_chunks // 2), unroll=8)
            def _(i):
                v = buf[pl.ds(i * L, L)]
                bits = lax.bitcast_convert_type(jnp.abs(v), jnp.int32)
                plsc.addupdate_scatter(
                    hist_v, [lax.shift_right_logical(bits, RSHIFT)], ones)

        def apply_half(r, h, cutoff, fuse_hist_buf):
            row_v = bufs[r]

            @plsc.parallel_loop(h * (n_chunks // 2),
                                (h + 1) * (n_chunks // 2), unroll=8)
            def _(i):
                v = row_v[pl.ds(i * L, L)]
                vb = lax.bitcast_convert_type(v, jnp.int32)
                ab = vb & jnp.int32(0x7FFFFFFF)
                a = lax.bitcast_convert_type(ab, jnp.float32)
                m = jnp.maximum(a - thr_v[pl.ds(i * L, L)],
                                jnp.float32(0.0))
                soft_b = (vb & jnp.int32(-0x80000000)) | \
                    lax.bitcast_convert_type(m, jnp.int32)
                out_b = jnp.where(ab >= cutoff, vb, soft_b)
                row_v[pl.ds(i * L, L)] = lax.bitcast_convert_type(
                    out_b, jnp.float32)
                if fuse_hist_buf is not None:
                    w = fuse_hist_buf[pl.ds(i * L, L)]
                    wbits = lax.bitcast_convert_type(jnp.abs(w), jnp.int32)
                    plsc.addupdate_scatter(
                        hist_v, [lax.shift_right_logical(wbits, RSHIFT)],
                        ones)

            out_copies.append(
                pltpu.async_copy(bufs[r].at[pl.ds(h * NH, NH)],
                                 out_hbm.at[rows[r], pl.ds(h * NH, NH)],
                                 out_sems[r][h]))

        for h in range(2):
            in_copies[0][h].wait()
            hist_half(bufs[0], h)
        cutoff0 = lax.shift_left(
            hist_scan(jnp.int32(k_top), re_zero=True), RSHIFT)
        thr_copy.wait()

        for h in range(2):
            in_copies[1][h].wait()
            apply_half(0, h, cutoff0, fuse_hist_buf=bufs[1])

        cutoff1 = lax.shift_left(
            hist_scan(jnp.int32(k_top), re_zero=False), RSHIFT)
        for h in range(2):
            apply_half(1, h, cutoff1, fuse_hist_buf=None)

        for c in out_copies:
            c.wait()

    return sc_kernel


def kernel(x, threshold):
    B, N = x.shape
    return _build(B, N)(x, threshold)
